# baseline (device time: 205524 ns/iter reference)
import jax
import jax.numpy as jnp
from jax import lax
from jax.experimental import pallas as pl
from jax.experimental.pallas import tpu as pltpu

N_DEV = 8
M_PER = 1024
K_PER = 1024
N_OUT = 4096
W_CHUNK = 128
CPO = K_PER // W_CHUNK
N_CHUNKS = CPO * N_DEV


def _fused(x_shard, w_mat):

    def body(x_ref, w_ref, o_ref, xg_ref, xs_ref, xstage_ref,
             wf_ref, wb_ref, wdma_sems, xdma_sems,
             send_sems, recv_sems):
        my = lax.axis_index("i")

        barrier_sem = pltpu.get_barrier_semaphore()
        for off in range(1, N_DEV):
            pl.semaphore_signal(
                barrier_sem, inc=1,
                device_id=(lax.rem(my + off, N_DEV),),
                device_id_type=pl.DeviceIdType.MESH,
            )
        pl.semaphore_wait(barrier_sem, N_DEV - 1)

        def w_chunk_copy(slot, t, c):
            src = lax.rem(my + (N_DEV - t), N_DEV)
            row0 = src * K_PER + c * W_CHUNK
            return pltpu.make_async_copy(
                w_ref.at[pl.ds(row0, W_CHUNK), :],
                wf_ref.at[slot],
                wdma_sems.at[slot],
            )

        w_chunk_copy(0, 0, 0).start()
        w_chunk_copy(1, 0, 1).start()

        def x_block_copy(slot, d):
            return pltpu.make_async_copy(
                x_ref.at[pl.ds(d * M_PER, M_PER), :],
                xstage_ref.at[slot],
                xdma_sems.at[slot],
            )

        x_block_copy(0, lax.rem(my + 1, N_DEV)).start()
        x_block_copy(1, lax.rem(my + 2, N_DEV)).start()
        rdmas = []
        for off in range(1, N_DEV):
            dst = lax.rem(my + off, N_DEV)
            x_block_copy((off - 1) % 2, dst).wait()
            xs_ref[off - 1] = xstage_ref[(off - 1) % 2].astype(jnp.bfloat16)
            nxt = off + 2
            if nxt <= N_DEV:
                x_block_copy((off - 1) % 2, lax.rem(my + nxt, N_DEV)).start()
            rdma = pltpu.make_async_remote_copy(
                src_ref=xs_ref.at[off - 1],
                dst_ref=xg_ref.at[off],
                send_sem=send_sems.at[off],
                recv_sem=recv_sems.at[off],
                device_id=(dst,),
                device_id_type=pl.DeviceIdType.MESH,
            )
            rdma.start()
            rdmas.append(rdma)

        x_block_copy(1, my).wait()
        xg_ref[0] = xstage_ref[1].astype(jnp.bfloat16)

        def origin_step(t, is_first):
            if not is_first:
                pltpu.make_async_remote_copy(
                    src_ref=xs_ref.at[0],
                    dst_ref=xg_ref.at[t],
                    send_sem=send_sems.at[0],
                    recv_sem=recv_sems.at[t],
                    device_id=(my,),
                    device_id_type=pl.DeviceIdType.MESH,
                ).wait_recv()
            for c in range(CPO):
                slot = c % 2
                w_chunk_copy(slot, t, c).wait()
                wb_ref[slot] = wf_ref[slot].astype(jnp.bfloat16)
                nt, nc = (t, c + 2) if c + 2 < CPO else (t + 1, c + 2 - CPO)
                if c + 2 < CPO:
                    w_chunk_copy(slot, nt, nc).start()
                else:
                    @pl.when(t != N_DEV - 1)
                    def _():
                        w_chunk_copy(slot, nt, nc).start()
                xb = xg_ref[t, :, c * W_CHUNK:(c + 1) * W_CHUNK]
                acc = jnp.dot(xb, wb_ref[slot],
                              preferred_element_type=jnp.float32)
                if is_first and c == 0:
                    o_ref[...] = acc
                else:
                    o_ref[...] += acc

        origin_step(0, True)

        def loop_body(t, _):
            origin_step(t, False)
            return ()

        lax.fori_loop(1, N_DEV, loop_body, (), unroll=False)

        for rdma in rdmas:
            rdma.wait_send()

    return pl.pallas_call(
        body,
        out_shape=jax.ShapeDtypeStruct((M_PER, N_OUT), jnp.float32),
        in_specs=[
            pl.BlockSpec(memory_space=pltpu.MemorySpace.HBM),
            pl.BlockSpec(memory_space=pltpu.MemorySpace.HBM),
        ],
        out_specs=pl.BlockSpec(memory_space=pltpu.VMEM),
        scratch_shapes=[
            pltpu.VMEM((N_DEV, M_PER, K_PER), jnp.bfloat16),
            pltpu.VMEM((N_DEV - 1, M_PER, K_PER), jnp.bfloat16),
            pltpu.VMEM((2, M_PER, K_PER), jnp.float32),
            pltpu.VMEM((2, W_CHUNK, N_OUT), jnp.float32),
            pltpu.VMEM((2, W_CHUNK, N_OUT), jnp.bfloat16),
            pltpu.SemaphoreType.DMA((2,)),
            pltpu.SemaphoreType.DMA((2,)),
            pltpu.SemaphoreType.DMA((N_DEV,)),
            pltpu.SemaphoreType.DMA((N_DEV,)),
        ],
        compiler_params=pltpu.CompilerParams(
            collective_id=0,
            vmem_limit_bytes=110 * 1024 * 1024,
        ),
    )(x_shard, w_mat)


def kernel(x, w_mat):
    return _fused(x, w_mat)


# device time: 200787 ns/iter; 1.0236x vs baseline; 1.0236x over previous
import jax
import jax.numpy as jnp
from jax import lax
from jax.experimental import pallas as pl
from jax.experimental.pallas import tpu as pltpu

N_DEV = 8
M_PER = 1024
K_PER = 1024
N_OUT = 4096
W_CHUNK = 256
CPO = K_PER // W_CHUNK
N_CHUNKS = CPO * N_DEV


def _fused(x_shard, w_mat):

    def body(x_ref, w_ref, o_ref, xg_ref, wf_ref, wb_ref,
             wdma_sems, send_sems, recv_sems):
        my = lax.axis_index("i")

        barrier_sem = pltpu.get_barrier_semaphore()
        for off in range(1, N_DEV):
            pl.semaphore_signal(
                barrier_sem, inc=1,
                device_id=(lax.rem(my + off, N_DEV),),
                device_id_type=pl.DeviceIdType.MESH,
            )
        pl.semaphore_wait(barrier_sem, N_DEV - 1)

        def w_chunk_copy(slot, t, c):
            src = lax.rem(my + (N_DEV - t), N_DEV)
            row0 = src * K_PER + c * W_CHUNK
            return pltpu.make_async_copy(
                w_ref.at[pl.ds(row0, W_CHUNK), :],
                wf_ref.at[slot],
                wdma_sems.at[slot],
            )

        w_chunk_copy(0, 0, 0).start()
        w_chunk_copy(1, 0, 1).start()

        rdmas = []
        for off in range(1, N_DEV):
            dst = lax.rem(my + off, N_DEV)
            rdma = pltpu.make_async_remote_copy(
                src_ref=x_ref.at[pl.ds(dst * M_PER, M_PER), :],
                dst_ref=xg_ref.at[off],
                send_sem=send_sems.at[off],
                recv_sem=recv_sems.at[off],
                device_id=(dst,),
                device_id_type=pl.DeviceIdType.MESH,
            )
            rdma.start()
            rdmas.append(rdma)

        xg_ref[0] = x_ref[pl.ds(my * M_PER, M_PER), :]

        def origin_step(t, is_first):
            if not is_first:
                pltpu.make_async_remote_copy(
                    src_ref=x_ref.at[pl.ds(0, M_PER), :],
                    dst_ref=xg_ref.at[t],
                    send_sem=send_sems.at[0],
                    recv_sem=recv_sems.at[t],
                    device_id=(my,),
                    device_id_type=pl.DeviceIdType.MESH,
                ).wait_recv()
            for c in range(CPO):
                slot = c % 2
                w_chunk_copy(slot, t, c).wait()
                wb_ref[slot] = wf_ref[slot].astype(jnp.bfloat16)
                nt, nc = (t, c + 2) if c + 2 < CPO else (t + 1, c + 2 - CPO)
                if c + 2 < CPO:
                    w_chunk_copy(slot, nt, nc).start()
                else:
                    @pl.when(t != N_DEV - 1)
                    def _():
                        w_chunk_copy(slot, nt, nc).start()
                xb = xg_ref[t, :, c * W_CHUNK:(c + 1) * W_CHUNK]
                acc = jnp.dot(xb, wb_ref[slot],
                              preferred_element_type=jnp.float32)
                if is_first and c == 0:
                    o_ref[...] = acc
                else:
                    o_ref[...] += acc

        origin_step(0, True)

        def loop_body(t, _):
            origin_step(t, False)
            return ()

        lax.fori_loop(1, N_DEV, loop_body, (), unroll=False)

        for rdma in rdmas:
            rdma.wait_send()

    return pl.pallas_call(
        body,
        out_shape=jax.ShapeDtypeStruct((M_PER, N_OUT), jnp.float32),
        in_specs=[
            pl.BlockSpec(memory_space=pltpu.VMEM),
            pl.BlockSpec(memory_space=pltpu.MemorySpace.HBM),
        ],
        out_specs=pl.BlockSpec(memory_space=pltpu.VMEM),
        scratch_shapes=[
            pltpu.VMEM((N_DEV, M_PER, K_PER), jnp.bfloat16),
            pltpu.VMEM((2, W_CHUNK, N_OUT), jnp.float32),
            pltpu.VMEM((2, W_CHUNK, N_OUT), jnp.bfloat16),
            pltpu.SemaphoreType.DMA((2,)),
            pltpu.SemaphoreType.DMA((N_DEV,)),
            pltpu.SemaphoreType.DMA((N_DEV,)),
        ],
        compiler_params=pltpu.CompilerParams(
            collective_id=0,
            vmem_limit_bytes=110 * 1024 * 1024,
        ),
    )(x_shard, w_mat)


def kernel(x, w_mat):
    x = x.astype(jnp.bfloat16)
    return _fused(x, w_mat)
